# baseline (speedup 1.0000x reference)
import functools

import jax
import jax.numpy as jnp
from jax import lax
from jax.experimental import pallas as pl
from jax.experimental.pallas import tpu as pltpu
from jax.experimental.pallas import tpu_sc as plsc

_HID = 64
_L = 200
_B = 4096
_VOCAB = 1000000
_NW = 32
_ROWS_PER_W = _B // _NW
_NBUF = 2
_SPLITS = ((0, 104), (104, 96))


def _tpe_body(x_hbm, tok_hbm, pos_hbm, out_hbm, *scratch):
  bufs = scratch[0:_NBUF]
  idxs = scratch[_NBUF:2 * _NBUF]
  pos_v = scratch[2 * _NBUF]
  isems = scratch[2 * _NBUF + 1:2 * _NBUF + 1 + _NBUF]
  gsems = scratch[2 * _NBUF + 1 + _NBUF:2 * _NBUF + 1 + 2 * _NBUF]
  ssems = scratch[2 * _NBUF + 1 + 2 * _NBUF:]

  wid = lax.axis_index("s") * 2 + lax.axis_index("c")
  row0 = wid * _ROWS_PER_W

  pltpu.sync_copy(pos_hbm.at[pl.ds(0, _L)], pos_v)

  def start_idx(ci, s):
    base = (row0 + ci) * _L
    pltpu.make_async_copy(
        x_hbm.at[pl.ds(base, _L)], idxs[s].at[pl.ds(0, _L)],
        isems[s]).start()

  def prep_and_gather(s):
    pltpu.make_async_copy(
        x_hbm.at[pl.ds(0, _L)], idxs[s].at[pl.ds(0, _L)],
        isems[s]).wait()

    @plsc.parallel_loop(0, _L, 16)
    def _(r):
      sl = pl.ds(r, 16)
      idxs[s][sl] = idxs[s][sl] * 2

    for (off, n) in _SPLITS:
      pltpu.make_async_copy(
          tok_hbm.at[idxs[s].at[pl.ds(off, n)]],
          bufs[s].at[pl.ds(off, n)],
          gsems[s],
      ).start()

  def wait_gather(s):
    pltpu.make_async_copy(
        tok_hbm.at[idxs[s].at[pl.ds(0, _L)]], bufs[s], gsems[s]).wait()

  def start_scatter(ci, s):
    pltpu.make_async_copy(
        bufs[s], out_hbm.at[row0 + ci, pl.ds(0, _L), pl.ds(0, _HID)],
        ssems[s]).start()

  def wait_scatter(s):
    pltpu.make_async_copy(
        bufs[s], out_hbm.at[0, pl.ds(0, _L), pl.ds(0, _HID)],
        ssems[s]).wait()

  def add_pos(s):
    buf = bufs[s]

    @plsc.parallel_loop(0, _L, 1, unroll=4)
    def _(r):
      for c in range(_HID // 16):
        sl = pl.ds(c * 16, 16)
        buf[r, sl] = buf[r, sl] + pos_v[r, sl]

  start_idx(0, 0)
  start_idx(1, 1)
  prep_and_gather(0)

  def step(i, carry):
    for k in range(_NBUF):
      ci = i * _NBUF + k

      @pl.when(ci + 1 < _ROWS_PER_W)
      def _():
        @pl.when(ci >= 1)
        def _():
          wait_scatter(1 - k)
        prep_and_gather(1 - k)

      wait_gather(k)

      @pl.when(ci + 2 < _ROWS_PER_W)
      def _():
        start_idx(ci + 2, k)

      add_pos(k)
      start_scatter(ci, k)
    return carry

  lax.fori_loop(0, _ROWS_PER_W // _NBUF, step, 0)

  for s in range(_NBUF):
    wait_scatter(s)


@jax.jit
def _tpe_call(x_flat, tok_rows, pos_table):
  mesh = plsc.VectorSubcoreMesh(core_axis_name="c", subcore_axis_name="s")
  kern = functools.partial(
      pl.kernel,
      mesh=mesh,
      compiler_params=pltpu.CompilerParams(use_tc_tiling_on_sc=False),
      out_type=jax.ShapeDtypeStruct((_B, _L, 2 * _HID), jnp.float32),
      scratch_types=(
          [pltpu.VMEM((_L, _HID), jnp.float32) for _ in range(_NBUF)]
          + [pltpu.VMEM((208,), jnp.int32) for _ in range(_NBUF)]
          + [pltpu.VMEM((_L, _HID), jnp.float32)]
          + [pltpu.SemaphoreType.DMA] * (3 * _NBUF)
      ),
  )(_tpe_body)
  return kern(x_flat, tok_rows, pos_table)


def kernel(x, tok_table, pos_table):
  x_flat = jnp.reshape(x.astype(jnp.int32), (_B * _L,))
  tok_rows = jnp.reshape(
      jnp.concatenate([tok_table, tok_table], axis=1), (2 * _VOCAB, _HID))
  out_wide = _tpe_call(x_flat, tok_rows, pos_table)
  return out_wide[:, :, :_HID]


# speedup vs baseline: 1.1660x; 1.1660x over previous
import functools

import jax
import jax.numpy as jnp
from jax import lax
from jax.experimental import pallas as pl
from jax.experimental.pallas import tpu as pltpu
from jax.experimental.pallas import tpu_sc as plsc

_HID = 64
_L = 200
_B = 4096
_VOCAB = 1000000
_NW = 32
_ROWS_PER_W = _B // _NW
_NBUF = 3
_SPLITS = ((0, 104), (104, 96))


def _tpe_body(x_hbm, tok_hbm, pos_hbm, out_hbm, *scratch):
  bufs = scratch[0:_NBUF]
  idxs = scratch[_NBUF:2 * _NBUF]
  pos_v = scratch[2 * _NBUF]
  isems = scratch[2 * _NBUF + 1:2 * _NBUF + 1 + _NBUF]
  gsems = scratch[2 * _NBUF + 1 + _NBUF:2 * _NBUF + 1 + 2 * _NBUF]
  ssems = scratch[2 * _NBUF + 1 + 2 * _NBUF:]

  wid = lax.axis_index("s") * 2 + lax.axis_index("c")
  row0 = wid * _ROWS_PER_W

  pltpu.sync_copy(pos_hbm.at[pl.ds(0, _L)], pos_v)

  def start_idx(ci, s):
    base = (row0 + ci) * _L
    pltpu.make_async_copy(
        x_hbm.at[pl.ds(base, _L)], idxs[s].at[pl.ds(0, _L)],
        isems[s]).start()

  def prep_and_gather(s):
    pltpu.make_async_copy(
        x_hbm.at[pl.ds(0, _L)], idxs[s].at[pl.ds(0, _L)],
        isems[s]).wait()

    @plsc.parallel_loop(0, _L, 16)
    def _(r):
      sl = pl.ds(r, 16)
      idxs[s][sl] = idxs[s][sl] * 2

    for (off, n) in _SPLITS:
      pltpu.make_async_copy(
          tok_hbm.at[idxs[s].at[pl.ds(off, n)]],
          bufs[s].at[pl.ds(off, n)],
          gsems[s],
      ).start()

  def wait_gather(s):
    pltpu.make_async_copy(
        tok_hbm.at[idxs[s].at[pl.ds(0, _L)]], bufs[s], gsems[s]).wait()

  def start_scatter(ci, s):
    pltpu.make_async_copy(
        bufs[s], out_hbm.at[row0 + ci, pl.ds(0, _L), pl.ds(0, _HID)],
        ssems[s]).start()

  def wait_scatter(s):
    pltpu.make_async_copy(
        bufs[s], out_hbm.at[0, pl.ds(0, _L), pl.ds(0, _HID)],
        ssems[s]).wait()

  def add_pos(s):
    buf = bufs[s]

    @plsc.parallel_loop(0, _L, 1, unroll=4)
    def _(r):
      for c in range(_HID // 16):
        sl = pl.ds(c * 16, 16)
        buf[r, sl] = buf[r, sl] + pos_v[r, sl]

  for j in range(_NBUF):
    start_idx(j, j)
  prep_and_gather(0)
  prep_and_gather(1)

  def chunk_tail(ci, k):
    wait_gather(k)

    @pl.when(ci + _NBUF < _ROWS_PER_W)
    def _():
      start_idx(ci + _NBUF, k)

    add_pos(k)
    start_scatter(ci, k)

  def step(i, carry):
    for k in range(_NBUF):
      ci = i * _NBUF + k
      s2 = (k + 2) % _NBUF

      @pl.when(ci + 2 < _ROWS_PER_W)
      def _():
        @pl.when(ci >= 1)
        def _():
          wait_scatter(s2)
        prep_and_gather(s2)

      chunk_tail(ci, k)
    return carry

  lax.fori_loop(0, (_ROWS_PER_W - 2) // _NBUF, step, 0)
  chunk_tail(_ROWS_PER_W - 2, (_ROWS_PER_W - 2) % _NBUF)
  chunk_tail(_ROWS_PER_W - 1, (_ROWS_PER_W - 1) % _NBUF)

  for s in range(_NBUF):
    wait_scatter(s)


@jax.jit
def _tpe_call(x_flat, tok_rows, pos_table):
  mesh = plsc.VectorSubcoreMesh(core_axis_name="c", subcore_axis_name="s")
  kern = functools.partial(
      pl.kernel,
      mesh=mesh,
      compiler_params=pltpu.CompilerParams(use_tc_tiling_on_sc=False),
      out_type=jax.ShapeDtypeStruct((_B, _L, 2 * _HID), jnp.float32),
      scratch_types=(
          [pltpu.VMEM((_L, _HID), jnp.float32) for _ in range(_NBUF)]
          + [pltpu.VMEM((208,), jnp.int32) for _ in range(_NBUF)]
          + [pltpu.VMEM((_L, _HID), jnp.float32)]
          + [pltpu.SemaphoreType.DMA] * (3 * _NBUF)
      ),
  )(_tpe_body)
  return kern(x_flat, tok_rows, pos_table)


def kernel(x, tok_table, pos_table):
  x_flat = jnp.reshape(x.astype(jnp.int32), (_B * _L,))
  tok_rows = jnp.reshape(
      jnp.pad(tok_table, ((0, 0), (0, _HID))), (2 * _VOCAB, _HID))
  out_wide = _tpe_call(x_flat, tok_rows, pos_table)
  return out_wide[:, :, :_HID]
